# baseline (speedup 1.0000x reference)
import jax
import jax.numpy as jnp
from jax import lax
from jax.experimental import pallas as pl
from jax.experimental.pallas import tpu as pltpu
from jax.experimental.pallas import tpu_sc as plsc

EMB = 64
PAIRW = 128
SCALE = 8.0

NUM_CORES = 2
NUM_SUBCORES = 16
NUM_WORKERS = NUM_CORES * NUM_SUBCORES

BATCH = 4096
SEQ = 200
BPW = BATCH // NUM_WORKERS
LANES = 16


def _body(table_hbm, tok_hbm, out_hbm,
          idx_all, gidx_all, gbuf0, gbuf1, tbuf0, tbuf1,
          gsem0, gsem1, osem0, osem1):
    gbuf = (gbuf0, gbuf1)
    tbuf = (tbuf0, tbuf1)
    gsem = (gsem0, gsem1)
    osem = (osem0, osem1)

    wid = lax.axis_index("s") * NUM_CORES + lax.axis_index("c")
    b0 = pl.multiple_of(wid * BPW, BPW)

    pltpu.sync_copy(tok_hbm.at[:, pl.ds(b0, BPW)], idx_all)

    def gix(r, c):
        for cc in range(BPW // LANES):
            sl = (r, pl.ds(cc * LANES, LANES))
            gidx_all[sl] = lax.shift_right_logical(idx_all[sl], 1)
        return c

    lax.fori_loop(0, SEQ, gix, 0, unroll=False)

    def fire_gather(h, e):
        pltpu.async_copy(table_hbm.at[gidx_all.at[h]], gbuf[e], gsem[e])

    def wait_gather(e):
        pltpu.make_async_copy(
            table_hbm.at[pl.ds(0, BPW)], gbuf[e], gsem[e]).wait()

    def fire_writeout(h, e):
        pltpu.async_copy(tbuf[e], out_hbm.at[h, :, pl.ds(b0, BPW)], osem[e])

    def wait_writeout(e):
        pltpu.make_async_copy(
            tbuf[e], out_hbm.at[0, :, pl.ds(0, BPW)], osem[e]).wait()

    row16 = [lax.iota(jnp.int32, LANES) + bc * LANES
             for bc in range(BPW // LANES)]

    def scale(h, e):
        def sbody(f4, c):
            cols = [(idx_all[h, pl.ds(bc * LANES, LANES)] & 1) * EMB
                    for bc in range(BPW // LANES)]
            for ff in range(4):
                f = f4 * 4 + ff
                for bc in range(BPW // LANES):
                    v = plsc.load_gather(gbuf[e], [row16[bc], cols[bc] + f])
                    tbuf[e][f, pl.ds(bc * LANES, LANES)] = v * SCALE
            return c

        lax.fori_loop(0, EMB // 4, sbody, 0, unroll=False)

    def pipe_step(h, e, first_round):
        wait_gather(e)
        fire_gather(h + 1, 1 - e)
        if not first_round:
            wait_writeout(e)
        scale(h, e)
        fire_writeout(h, e)

    fire_gather(0, 0)
    pipe_step(0, 0, True)
    pipe_step(1, 1, True)

    def loop_body(i, c):
        h = i * 2
        pipe_step(h, 0, False)
        pipe_step(h + 1, 1, False)
        return c

    lax.fori_loop(1, (SEQ - 2) // 2, loop_body, 0, unroll=False)

    h = SEQ - 2
    wait_gather(0)
    fire_gather(h + 1, 1)
    wait_writeout(0)
    scale(h, 0)
    fire_writeout(h, 0)

    wait_gather(1)
    wait_writeout(1)
    scale(h + 1, 1)
    fire_writeout(h + 1, 1)

    wait_writeout(0)
    wait_writeout(1)


def kernel(tokens, table):
    b, s = tokens.shape
    tpack = table.reshape(table.shape[0] // 2, PAIRW)
    tok_t = tokens.astype(jnp.int32).T

    mesh = plsc.VectorSubcoreMesh(
        core_axis_name="c", subcore_axis_name="s",
        num_cores=NUM_CORES, num_subcores=NUM_SUBCORES,
    )
    out_t = pl.kernel(
        _body,
        out_type=jax.ShapeDtypeStruct((s, EMB, b), jnp.float32),
        mesh=mesh,
        compiler_params=pltpu.CompilerParams(
            use_tc_tiling_on_sc=True, needs_layout_passes=False),
        scratch_types=[
            pltpu.VMEM((SEQ, BPW), jnp.int32),
            pltpu.VMEM((SEQ, BPW), jnp.int32),
            pltpu.VMEM((BPW, PAIRW), jnp.float32),
            pltpu.VMEM((BPW, PAIRW), jnp.float32),
            pltpu.VMEM((EMB, BPW), jnp.float32),
            pltpu.VMEM((EMB, BPW), jnp.float32),
            pltpu.SemaphoreType.DMA,
            pltpu.SemaphoreType.DMA,
            pltpu.SemaphoreType.DMA,
            pltpu.SemaphoreType.DMA,
        ],
    )(tpack, tok_t)
    return out_t.transpose(2, 0, 1)


# speedup vs baseline: 2.3746x; 2.3746x over previous
import jax
import jax.numpy as jnp
from jax import lax
from jax.experimental import pallas as pl
from jax.experimental.pallas import tpu as pltpu
from jax.experimental.pallas import tpu_sc as plsc

EMB = 64
PADW = 128
SCALE = 8.0
VOCAB = 1000000

NUM_CORES = 2
NUM_SUBCORES = 16
NUM_WORKERS = NUM_CORES * NUM_SUBCORES

N_TOK = 4096 * 200
TOK_PER_W = N_TOK // NUM_WORKERS
STEP = 128
STEPS = TOK_PER_W // STEP
IDX_ROWS = STEPS

VBLK = 6400


def _widen_body(tt_ref, out_ref):
    out_ref[:, :EMB] = jnp.transpose(tt_ref[...]) * SCALE


def _widen(table_t):
    return pl.pallas_call(
        _widen_body,
        grid=((VOCAB + VBLK - 1) // VBLK,),
        in_specs=[pl.BlockSpec((EMB, VBLK), lambda i: (0, i))],
        out_specs=pl.BlockSpec((VBLK, PADW), lambda i: (i, 0)),
        out_shape=jax.ShapeDtypeStruct((VOCAB, PADW), jnp.float32),
    )(table_t)


def _gather_body(table_hbm, tok_hbm, out_hbm,
                 idx_all, gbuf0, gbuf1, obuf0, obuf1,
                 gsem0, gsem1, osem0, osem1):
    gbuf = (gbuf0, gbuf1)
    obuf = (obuf0, obuf1)
    gsem = (gsem0, gsem1)
    osem = (osem0, osem1)

    wid = lax.axis_index("s") * NUM_CORES + lax.axis_index("c")
    base_tok = wid * TOK_PER_W
    base_row = base_tok // STEP

    pltpu.sync_copy(tok_hbm.at[pl.ds(pl.multiple_of(base_row, 8), IDX_ROWS)],
                    idx_all)

    def fire_gather(h, e):
        pltpu.async_copy(table_hbm.at[idx_all.at[h]], gbuf[e], gsem[e])

    def wait_gather(e):
        pltpu.make_async_copy(
            table_hbm.at[pl.ds(0, STEP)], gbuf[e], gsem[e]).wait()

    def fire_writeout(h, e):
        tok0 = pl.multiple_of(base_tok + h * STEP, 8)
        pltpu.async_copy(obuf[e], out_hbm.at[pl.ds(tok0, STEP)], osem[e])

    def wait_writeout(e):
        pltpu.make_async_copy(
            obuf[e], out_hbm.at[pl.ds(0, STEP)], osem[e]).wait()

    def repack(e):
        def sbody(r, c):
            for tt in range(4):
                for cc in range(EMB // 16):
                    sl = (r * 4 + tt, pl.ds(cc * 16, 16))
                    obuf[e][sl] = gbuf[e][sl]
            return c

        lax.fori_loop(0, STEP // 4, sbody, 0, unroll=False)

    def pipe_step(h, e, first_round):
        wait_gather(e)
        fire_gather(h + 1, 1 - e)
        if not first_round:
            wait_writeout(e)
        repack(e)
        fire_writeout(h, e)

    fire_gather(0, 0)
    pipe_step(0, 0, True)
    pipe_step(1, 1, True)

    def loop_body(i, c):
        h = i * 2
        pipe_step(h, 0, False)
        pipe_step(h + 1, 1, False)
        return c

    lax.fori_loop(1, (STEPS - 2) // 2, loop_body, 0, unroll=False)

    h = STEPS - 2
    wait_gather(0)
    fire_gather(h + 1, 1)
    wait_writeout(0)
    repack(0)
    fire_writeout(h, 0)

    wait_gather(1)
    wait_writeout(1)
    repack(1)
    fire_writeout(h + 1, 1)

    wait_writeout(0)
    wait_writeout(1)


def kernel(tokens, table):
    b, s = tokens.shape
    n = b * s
    twide = _widen(table.T)
    tok2d = tokens.astype(jnp.int32).reshape(n // STEP, STEP)

    mesh = plsc.VectorSubcoreMesh(
        core_axis_name="c", subcore_axis_name="s",
        num_cores=NUM_CORES, num_subcores=NUM_SUBCORES,
    )
    out = pl.kernel(
        _gather_body,
        out_type=jax.ShapeDtypeStruct((n, EMB), jnp.float32),
        mesh=mesh,
        compiler_params=pltpu.CompilerParams(use_tc_tiling_on_sc=True),
        scratch_types=[
            pltpu.VMEM((IDX_ROWS, STEP), jnp.int32),
            pltpu.VMEM((STEP, PADW), jnp.float32),
            pltpu.VMEM((STEP, PADW), jnp.float32),
            pltpu.VMEM((STEP, EMB), jnp.float32),
            pltpu.VMEM((STEP, EMB), jnp.float32),
            pltpu.SemaphoreType.DMA,
            pltpu.SemaphoreType.DMA,
            pltpu.SemaphoreType.DMA,
            pltpu.SemaphoreType.DMA,
        ],
    )(twide, tok2d)
    return out.reshape(b, s, EMB)


# speedup vs baseline: 2.5291x; 1.0650x over previous
import jax
import jax.numpy as jnp
from jax import lax
from jax.experimental import pallas as pl
from jax.experimental.pallas import tpu as pltpu
from jax.experimental.pallas import tpu_sc as plsc

EMB = 64
PADW = 128
SCALE = 8.0
VOCAB = 1000000

NUM_CORES = 2
NUM_SUBCORES = 16
NUM_WORKERS = NUM_CORES * NUM_SUBCORES

N_TOK = 4096 * 200
TOK_PER_W = N_TOK // NUM_WORKERS
STEP = 256
IDXW = 128
STEPS = TOK_PER_W // STEP
IDX_ROWS = TOK_PER_W // IDXW
NBUF = 3

VBLK = 6400


def _widen_body(tt_ref, out_ref):
    out_ref[:, :EMB] = jnp.transpose(tt_ref[...]) * SCALE


def _widen(table_t):
    return pl.pallas_call(
        _widen_body,
        grid=((VOCAB + VBLK - 1) // VBLK,),
        in_specs=[pl.BlockSpec((EMB, VBLK), lambda i: (0, i))],
        out_specs=pl.BlockSpec((VBLK, PADW), lambda i: (i, 0)),
        out_shape=jax.ShapeDtypeStruct((VOCAB, PADW), jnp.float32),
    )(table_t)


def _gather_body(table_hbm, tok_hbm, out_hbm,
                 idx_all, gbuf0, gbuf1, gbuf2,
                 gsem0, gsem1, gsem2, osem0, osem1, osem2):
    gbuf = (gbuf0, gbuf1, gbuf2)
    gsem = (gsem0, gsem1, gsem2)
    osem = (osem0, osem1, osem2)

    wid = lax.axis_index("s") * NUM_CORES + lax.axis_index("c")
    base_tok = wid * TOK_PER_W
    base_row = base_tok // IDXW

    pltpu.sync_copy(tok_hbm.at[pl.ds(pl.multiple_of(base_row, 8), IDX_ROWS)],
                    idx_all)

    def fire_gather(h, d):
        for j in range(STEP // IDXW):
            pltpu.async_copy(
                table_hbm.at[idx_all.at[h * (STEP // IDXW) + j]],
                gbuf[d].at[pl.ds(j * IDXW, IDXW)],
                gsem[d],
            )

    def wait_gather(d):
        pltpu.make_async_copy(
            table_hbm.at[pl.ds(0, STEP)], gbuf[d], gsem[d]).wait()

    def fire_writeout(h, d):
        tok0 = pl.multiple_of(base_tok + h * STEP, 8)
        pltpu.async_copy(gbuf[d], out_hbm.at[pl.ds(tok0, STEP)], osem[d])

    def wait_writeout(d):
        pltpu.make_async_copy(
            gbuf[d], out_hbm.at[pl.ds(0, STEP)], osem[d]).wait()

    def pipe_step(h, d, first_round):
        wait_gather(d)
        fire_writeout(h, d)
        d2 = (d + 1) % NBUF
        if not first_round:
            wait_writeout(d2)
        fire_gather(h + 1, d2)

    fire_gather(0, 0)
    pipe_step(0, 0, True)
    pipe_step(1, 1, True)
    pipe_step(2, 2, False)

    def loop_body(i, c):
        h = i * NBUF
        pipe_step(h, 0, False)
        pipe_step(h + 1, 1, False)
        pipe_step(h + 2, 2, False)
        return c

    lax.fori_loop(1, 1 + (STEPS - 1 - NBUF) // NBUF, loop_body, 0,
                  unroll=False)

    h = STEPS - 1
    wait_gather(0)
    fire_writeout(h, 0)

    wait_writeout(1)
    wait_writeout(2)
    wait_writeout(0)


def kernel(tokens, table):
    b, s = tokens.shape
    n = b * s
    twide = _widen(table.T)
    tok2d = tokens.astype(jnp.int32).reshape(n // IDXW, IDXW)

    mesh = plsc.VectorSubcoreMesh(
        core_axis_name="c", subcore_axis_name="s",
        num_cores=NUM_CORES, num_subcores=NUM_SUBCORES,
    )
    out = pl.kernel(
        _gather_body,
        out_type=jax.ShapeDtypeStruct((n, PADW), jnp.float32),
        mesh=mesh,
        compiler_params=pltpu.CompilerParams(use_tc_tiling_on_sc=True),
        scratch_types=[
            pltpu.VMEM((IDX_ROWS, IDXW), jnp.int32),
            pltpu.VMEM((STEP, PADW), jnp.float32),
            pltpu.VMEM((STEP, PADW), jnp.float32),
            pltpu.VMEM((STEP, PADW), jnp.float32),
            pltpu.SemaphoreType.DMA,
            pltpu.SemaphoreType.DMA,
            pltpu.SemaphoreType.DMA,
            pltpu.SemaphoreType.DMA,
            pltpu.SemaphoreType.DMA,
            pltpu.SemaphoreType.DMA,
        ],
    )(twide, tok2d)
    return out.reshape(b, s, PADW)[:, :, :EMB]


# speedup vs baseline: 2.6493x; 1.0475x over previous
import jax
import jax.numpy as jnp
from jax import lax
from jax.experimental import pallas as pl
from jax.experimental.pallas import tpu as pltpu
from jax.experimental.pallas import tpu_sc as plsc

EMB = 64
PADW = 128
SCALE = 8.0
VOCAB = 1000000

NUM_CORES = 2
NUM_SUBCORES = 16
NUM_WORKERS = NUM_CORES * NUM_SUBCORES

N_TOK = 4096 * 200
TOK_PER_W = N_TOK // NUM_WORKERS
STEP = 256
IDXW = 128
STEPS = TOK_PER_W // STEP
IDX_ROWS = TOK_PER_W // IDXW
NBUF = 3

VBLK = 12800


def _widen_body(tt_ref, out_ref):
    out_ref[:, :EMB] = jnp.transpose(tt_ref[...]) * SCALE


def _widen(table_t):
    return pl.pallas_call(
        _widen_body,
        grid=((VOCAB + VBLK - 1) // VBLK,),
        in_specs=[pl.BlockSpec((EMB, VBLK), lambda i: (0, i))],
        out_specs=pl.BlockSpec((VBLK, PADW), lambda i: (i, 0)),
        out_shape=jax.ShapeDtypeStruct((VOCAB, PADW), jnp.float32),
    )(table_t)


def _gather_body(table_hbm, tok_hbm, out_hbm,
                 idx_all, gbuf0, gbuf1, gbuf2,
                 gsem0, gsem1, gsem2, osem0, osem1, osem2):
    gbuf = (gbuf0, gbuf1, gbuf2)
    gsem = (gsem0, gsem1, gsem2)
    osem = (osem0, osem1, osem2)

    wid = lax.axis_index("s") * NUM_CORES + lax.axis_index("c")
    base_tok = wid * TOK_PER_W
    base_row = base_tok // IDXW

    pltpu.sync_copy(tok_hbm.at[pl.ds(pl.multiple_of(base_row, 8), IDX_ROWS)],
                    idx_all)

    def fire_gather(h, d):
        for j in range(STEP // IDXW):
            pltpu.async_copy(
                table_hbm.at[idx_all.at[h * (STEP // IDXW) + j]],
                gbuf[d].at[pl.ds(j * IDXW, IDXW)],
                gsem[d],
            )

    def wait_gather(d):
        pltpu.make_async_copy(
            table_hbm.at[pl.ds(0, STEP)], gbuf[d], gsem[d]).wait()

    def fire_writeout(h, d):
        tok0 = pl.multiple_of(base_tok + h * STEP, 8)
        pltpu.async_copy(gbuf[d], out_hbm.at[pl.ds(tok0, STEP)], osem[d])

    def wait_writeout(d):
        pltpu.make_async_copy(
            gbuf[d], out_hbm.at[pl.ds(0, STEP)], osem[d]).wait()

    def pipe_step(h, d, first_round):
        wait_gather(d)
        fire_writeout(h, d)
        d2 = (d + 1) % NBUF
        if not first_round:
            wait_writeout(d2)
        fire_gather(h + 1, d2)

    fire_gather(0, 0)
    pipe_step(0, 0, True)
    pipe_step(1, 1, True)
    pipe_step(2, 2, False)

    def loop_body(i, c):
        h = i * NBUF
        pipe_step(h, 0, False)
        pipe_step(h + 1, 1, False)
        pipe_step(h + 2, 2, False)
        return c

    lax.fori_loop(1, 1 + (STEPS - 1 - NBUF) // NBUF, loop_body, 0,
                  unroll=False)

    h = STEPS - 1
    wait_gather(0)
    fire_writeout(h, 0)

    wait_writeout(1)
    wait_writeout(2)
    wait_writeout(0)


def kernel(tokens, table):
    b, s = tokens.shape
    n = b * s
    twide = _widen(table.T)
    tok2d = tokens.astype(jnp.int32).reshape(n // IDXW, IDXW)

    mesh = plsc.VectorSubcoreMesh(
        core_axis_name="c", subcore_axis_name="s",
        num_cores=NUM_CORES, num_subcores=NUM_SUBCORES,
    )
    out = pl.kernel(
        _gather_body,
        out_type=jax.ShapeDtypeStruct((n, PADW), jnp.float32),
        mesh=mesh,
        compiler_params=pltpu.CompilerParams(use_tc_tiling_on_sc=True),
        scratch_types=[
            pltpu.VMEM((IDX_ROWS, IDXW), jnp.int32),
            pltpu.VMEM((STEP, PADW), jnp.float32),
            pltpu.VMEM((STEP, PADW), jnp.float32),
            pltpu.VMEM((STEP, PADW), jnp.float32),
            pltpu.SemaphoreType.DMA,
            pltpu.SemaphoreType.DMA,
            pltpu.SemaphoreType.DMA,
            pltpu.SemaphoreType.DMA,
            pltpu.SemaphoreType.DMA,
            pltpu.SemaphoreType.DMA,
        ],
    )(twide, tok2d)
    return out.reshape(b, s, PADW)[:, :, :EMB]


# speedup vs baseline: 2.6774x; 1.0106x over previous
import jax
import jax.numpy as jnp
from jax import lax
from jax.experimental import pallas as pl
from jax.experimental.pallas import tpu as pltpu
from jax.experimental.pallas import tpu_sc as plsc

EMB = 64
PADW = 128
SCALE = 8.0
VOCAB = 1000000

NUM_CORES = 2
NUM_SUBCORES = 16
NUM_WORKERS = NUM_CORES * NUM_SUBCORES

N_TOK = 4096 * 200
TOK_PER_W = N_TOK // NUM_WORKERS
STEP = 256
IDXW = 128
STEPS = TOK_PER_W // STEP
IDX_ROWS = TOK_PER_W // IDXW
NBUF = 3

VBLK = 25600


def _widen_body(tt_ref, out_ref):
    out_ref[:, :EMB] = jnp.transpose(tt_ref[...]) * SCALE


def _widen(table_t):
    return pl.pallas_call(
        _widen_body,
        grid=((VOCAB + VBLK - 1) // VBLK,),
        in_specs=[pl.BlockSpec((EMB, VBLK), lambda i: (0, i))],
        out_specs=pl.BlockSpec((VBLK, PADW), lambda i: (i, 0)),
        out_shape=jax.ShapeDtypeStruct((VOCAB, PADW), jnp.float32),
    )(table_t)


def _gather_body(table_hbm, tok_hbm, out_hbm,
                 idx_all, gbuf0, gbuf1, gbuf2,
                 gsem0, gsem1, gsem2, osem0, osem1, osem2):
    gbuf = (gbuf0, gbuf1, gbuf2)
    gsem = (gsem0, gsem1, gsem2)
    osem = (osem0, osem1, osem2)

    wid = lax.axis_index("s") * NUM_CORES + lax.axis_index("c")
    base_tok = wid * TOK_PER_W
    base_row = base_tok // IDXW

    pltpu.sync_copy(tok_hbm.at[pl.ds(pl.multiple_of(base_row, 8), IDX_ROWS)],
                    idx_all)

    def fire_gather(h, d):
        for j in range(STEP // IDXW):
            pltpu.async_copy(
                table_hbm.at[idx_all.at[h * (STEP // IDXW) + j]],
                gbuf[d].at[pl.ds(j * IDXW, IDXW)],
                gsem[d],
            )

    def wait_gather(d):
        pltpu.make_async_copy(
            table_hbm.at[pl.ds(0, STEP)], gbuf[d], gsem[d]).wait()

    def fire_writeout(h, d):
        tok0 = pl.multiple_of(base_tok + h * STEP, 8)
        pltpu.async_copy(gbuf[d], out_hbm.at[pl.ds(tok0, STEP)], osem[d])

    def wait_writeout(d):
        pltpu.make_async_copy(
            gbuf[d], out_hbm.at[pl.ds(0, STEP)], osem[d]).wait()

    def pipe_step(h, d, first_round):
        wait_gather(d)
        fire_writeout(h, d)
        d2 = (d + 1) % NBUF
        if not first_round:
            wait_writeout(d2)
        fire_gather(h + 1, d2)

    fire_gather(0, 0)
    pipe_step(0, 0, True)
    pipe_step(1, 1, True)
    pipe_step(2, 2, False)

    def loop_body(i, c):
        h = i * NBUF
        pipe_step(h, 0, False)
        pipe_step(h + 1, 1, False)
        pipe_step(h + 2, 2, False)
        return c

    lax.fori_loop(1, 1 + (STEPS - 1 - NBUF) // NBUF, loop_body, 0,
                  unroll=False)

    h = STEPS - 1
    wait_gather(0)
    fire_writeout(h, 0)

    wait_writeout(1)
    wait_writeout(2)
    wait_writeout(0)


def kernel(tokens, table):
    b, s = tokens.shape
    n = b * s
    twide = _widen(table.T)
    tok2d = tokens.astype(jnp.int32).reshape(n // IDXW, IDXW)

    mesh = plsc.VectorSubcoreMesh(
        core_axis_name="c", subcore_axis_name="s",
        num_cores=NUM_CORES, num_subcores=NUM_SUBCORES,
    )
    out = pl.kernel(
        _gather_body,
        out_type=jax.ShapeDtypeStruct((n, PADW), jnp.float32),
        mesh=mesh,
        compiler_params=pltpu.CompilerParams(use_tc_tiling_on_sc=True),
        scratch_types=[
            pltpu.VMEM((IDX_ROWS, IDXW), jnp.int32),
            pltpu.VMEM((STEP, PADW), jnp.float32),
            pltpu.VMEM((STEP, PADW), jnp.float32),
            pltpu.VMEM((STEP, PADW), jnp.float32),
            pltpu.SemaphoreType.DMA,
            pltpu.SemaphoreType.DMA,
            pltpu.SemaphoreType.DMA,
            pltpu.SemaphoreType.DMA,
            pltpu.SemaphoreType.DMA,
            pltpu.SemaphoreType.DMA,
        ],
    )(twide, tok2d)
    return out.reshape(b, s, PADW)[:, :, :EMB]
